# baseline (speedup 1.0000x reference)
import functools

import jax
import jax.numpy as jnp
from jax import lax
from jax.experimental import pallas as pl
from jax.experimental.pallas import tpu as pltpu
from jax.experimental.pallas import tpu_sc as plsc

NC = 2
NS = 16
NW = NC * NS


def _sc_gather_logits(node_flat, b_of_i, r32, m32, p32, bg_flat, meta_flat,
                      pers_flat, B, L, R, V):
    b_per_w = B // NW
    n_per_w = b_per_w * L
    c16 = n_per_w // 16
    cdma = n_per_w // 128
    KW = 4

    mesh = plsc.VectorSubcoreMesh(core_axis_name="c", subcore_axis_name="s",
                                  num_cores=NC, num_subcores=NS)

    @functools.partial(
        pl.kernel,
        out_type=jax.ShapeDtypeStruct((B * L,), jnp.float32),
        mesh=mesh,
        compiler_params=pltpu.CompilerParams(needs_layout_passes=False),
        scratch_types=[
            pltpu.VMEM((n_per_w,), jnp.int32),
            pltpu.VMEM((n_per_w,), jnp.int32),
            pltpu.VMEM((b_per_w,), jnp.int32),
            pltpu.VMEM((b_per_w,), jnp.int32),
            pltpu.VMEM((b_per_w,), jnp.int32),
            pltpu.VMEM((b_per_w,), jnp.int32),
            pltpu.VMEM((b_per_w,), jnp.int32),
            pltpu.VMEM((b_per_w,), jnp.int32),
            pltpu.VMEM((n_per_w,), jnp.int32),
            pltpu.VMEM((n_per_w,), jnp.int32),
            pltpu.VMEM((n_per_w,), jnp.int32),
            pltpu.VMEM((n_per_w,), jnp.float32),
            pltpu.VMEM((n_per_w,), jnp.float32),
            pltpu.VMEM((n_per_w,), jnp.float32),
            pltpu.SemaphoreType.DMA,
        ],
    )
    def k(node_hbm, b_hbm, r_hbm, m_hbm, p_hbm, bg_hbm, meta_hbm, pers_hbm,
          out_hbm, b_v, node_v, r_v, m_v, p_v, bgb_v, mb_v, pb_v,
          bg_i, me_i, pe_i, bg_v, me_v, pe_v, sem):
        wid = lax.axis_index("s") * NC + lax.axis_index("c")
        eb = pl.multiple_of(wid * b_per_w, 8)
        en = pl.multiple_of(wid * n_per_w, 8)

        pltpu.sync_copy(b_hbm, b_v)
        pltpu.sync_copy(node_hbm.at[pl.ds(en, n_per_w)], node_v)
        pltpu.sync_copy(r_hbm.at[pl.ds(eb, b_per_w)], r_v)
        pltpu.sync_copy(m_hbm.at[pl.ds(eb, b_per_w)], m_v)
        pltpu.sync_copy(p_hbm.at[pl.ds(eb, b_per_w)], p_v)

        def base_body(j, carry):
            off = pl.multiple_of(j * 16, 16)
            r16 = r_v[pl.ds(off, 16)]
            m16 = m_v[pl.ds(off, 16)]
            p16 = p_v[pl.ds(off, 16)]
            bgb_v[pl.ds(off, 16)] = r16 * V
            mb_v[pl.ds(off, 16)] = (m16 * R + r16) * V
            pb_v[pl.ds(off, 16)] = (p16 * R + r16) * V
            return carry
        lax.fori_loop(0, b_per_w // 16, base_body, 0)

        def idx_body(c, carry):
            off = pl.multiple_of(c * 16, 16)
            b16 = b_v[pl.ds(off, 16)]
            node16 = node_v[pl.ds(off, 16)]
            bgb = plsc.load_gather(bgb_v, [b16])
            mb = plsc.load_gather(mb_v, [b16])
            pb = plsc.load_gather(pb_v, [b16])
            bg_i[pl.ds(off, 16)] = bgb + node16
            me_i[pl.ds(off, 16)] = mb + node16
            pe_i[pl.ds(off, 16)] = pb + node16
            return carry
        lax.fori_loop(0, c16, idx_body, 0)

        def dma_body(g, carry):
            base = pl.multiple_of(g * (KW * 128), 128)
            cps = []
            for t in range(KW):
                off = base + t * 128
                cps.append(pltpu.async_copy(
                    bg_hbm.at[bg_i.at[pl.ds(off, 128)]],
                    bg_v.at[pl.ds(off, 128)], sem))
                cps.append(pltpu.async_copy(
                    meta_hbm.at[me_i.at[pl.ds(off, 128)]],
                    me_v.at[pl.ds(off, 128)], sem))
                cps.append(pltpu.async_copy(
                    pers_hbm.at[pe_i.at[pl.ds(off, 128)]],
                    pe_v.at[pl.ds(off, 128)], sem))
            for cp in cps:
                cp.wait()
            return carry
        lax.fori_loop(0, cdma // KW, dma_body, 0)

        def sum_body(c, carry):
            off = pl.multiple_of(c * 16, 16)
            bg_v[pl.ds(off, 16)] = (bg_v[pl.ds(off, 16)]
                                    + me_v[pl.ds(off, 16)]
                                    + pe_v[pl.ds(off, 16)])
            return carry
        lax.fori_loop(0, c16, sum_body, 0)

        pltpu.sync_copy(bg_v, out_hbm.at[pl.ds(en, n_per_w)])

    return k(node_flat, b_of_i, r32, m32, p32, bg_flat, meta_flat, pers_flat)


def _tc_logsigmoid_reduce(z, signs, paths, pad):
    B, L = z.shape
    blk = 1024

    def body(z_ref, s_ref, p_ref, o_ref):
        x = s_ref[...] * z_ref[...]
        lp = jnp.minimum(x, 0.0) - jnp.log1p(jnp.exp(-jnp.abs(x)))
        lp = jnp.where(p_ref[...] != pad, lp, 0.0)
        o_ref[...] = jnp.sum(lp, axis=1)

    return pl.pallas_call(
        body,
        grid=(B // blk,),
        in_specs=[
            pl.BlockSpec((blk, L), lambda i: (i, 0)),
            pl.BlockSpec((blk, L), lambda i: (i, 0)),
            pl.BlockSpec((blk, L), lambda i: (i, 0)),
        ],
        out_specs=pl.BlockSpec((blk,), lambda i: (i,)),
        out_shape=jax.ShapeDtypeStruct((B,), jnp.float32),
    )(z, signs, paths)


def kernel(m_idx, p_idx, r_idx, node_paths, node_signs, eta_bg, eta_meta, eta_pers):
    B, L = node_paths.shape
    R, V = eta_bg.shape
    paths32 = node_paths.astype(jnp.int32)
    n_per_w = (B // NW) * L
    b_of_i = (jnp.arange(n_per_w, dtype=jnp.int32) // L).astype(jnp.int32)
    z = _sc_gather_logits(
        paths32.reshape(-1),
        b_of_i,
        r_idx.astype(jnp.int32),
        m_idx.astype(jnp.int32),
        p_idx.astype(jnp.int32),
        eta_bg.reshape(-1),
        eta_meta.reshape(-1),
        eta_pers.reshape(-1),
        B, L, R, V)
    return _tc_logsigmoid_reduce(z.reshape(B, L), node_signs, paths32, V - 1)


# speedup vs baseline: 1.0040x; 1.0040x over previous
import functools

import jax
import jax.numpy as jnp
from jax import lax
from jax.experimental import pallas as pl
from jax.experimental.pallas import tpu as pltpu
from jax.experimental.pallas import tpu_sc as plsc

NC = 2
NS = 16
NW = NC * NS


def _sc_gather_logits(node_flat, b_of_i, r32, m32, p32, bg_flat, meta_flat,
                      pers_flat, B, L, R, V):
    b_per_w = B // NW
    n_per_w = b_per_w * L
    c16 = n_per_w // 16
    cdma = n_per_w // 128
    KW = 4

    mesh = plsc.VectorSubcoreMesh(core_axis_name="c", subcore_axis_name="s",
                                  num_cores=NC, num_subcores=NS)

    @functools.partial(
        pl.kernel,
        out_type=jax.ShapeDtypeStruct((B * L,), jnp.float32),
        mesh=mesh,
        compiler_params=pltpu.CompilerParams(needs_layout_passes=False),
        scratch_types=[
            pltpu.VMEM((n_per_w,), jnp.int32),
            pltpu.VMEM((n_per_w,), jnp.int32),
            pltpu.VMEM((b_per_w,), jnp.int32),
            pltpu.VMEM((b_per_w,), jnp.int32),
            pltpu.VMEM((b_per_w,), jnp.int32),
            pltpu.VMEM((b_per_w,), jnp.int32),
            pltpu.VMEM((b_per_w,), jnp.int32),
            pltpu.VMEM((b_per_w,), jnp.int32),
            pltpu.VMEM((n_per_w,), jnp.int32),
            pltpu.VMEM((n_per_w,), jnp.int32),
            pltpu.VMEM((n_per_w,), jnp.int32),
            pltpu.VMEM((n_per_w,), jnp.float32),
            pltpu.VMEM((n_per_w,), jnp.float32),
            pltpu.VMEM((n_per_w,), jnp.float32),
            pltpu.SemaphoreType.DMA,
        ],
    )
    def k(node_hbm, b_hbm, r_hbm, m_hbm, p_hbm, bg_hbm, meta_hbm, pers_hbm,
          out_hbm, b_v, node_v, r_v, m_v, p_v, bgb_v, mb_v, pb_v,
          bg_i, me_i, pe_i, bg_v, me_v, pe_v, sem):
        wid = lax.axis_index("s") * NC + lax.axis_index("c")
        eb = pl.multiple_of(wid * b_per_w, 8)
        en = pl.multiple_of(wid * n_per_w, 8)

        pltpu.sync_copy(b_hbm, b_v)
        pltpu.sync_copy(node_hbm.at[pl.ds(en, n_per_w)], node_v)
        pltpu.sync_copy(r_hbm.at[pl.ds(eb, b_per_w)], r_v)
        pltpu.sync_copy(m_hbm.at[pl.ds(eb, b_per_w)], m_v)
        pltpu.sync_copy(p_hbm.at[pl.ds(eb, b_per_w)], p_v)

        def base_body(j, carry):
            off = pl.multiple_of(j * 16, 16)
            r16 = r_v[pl.ds(off, 16)]
            m16 = m_v[pl.ds(off, 16)]
            p16 = p_v[pl.ds(off, 16)]
            bgb_v[pl.ds(off, 16)] = r16 * V
            mb_v[pl.ds(off, 16)] = (m16 * R + r16) * V
            pb_v[pl.ds(off, 16)] = (p16 * R + r16) * V
            return carry
        lax.fori_loop(0, b_per_w // 16, base_body, 0)

        def idx_body(c, carry):
            off = pl.multiple_of(c * 16, 16)
            b16 = b_v[pl.ds(off, 16)]
            node16 = node_v[pl.ds(off, 16)]
            bgb = plsc.load_gather(bgb_v, [b16])
            mb = plsc.load_gather(mb_v, [b16])
            pb = plsc.load_gather(pb_v, [b16])
            bg_i[pl.ds(off, 16)] = bgb + node16
            me_i[pl.ds(off, 16)] = mb + node16
            pe_i[pl.ds(off, 16)] = pb + node16
            return carry
        lax.fori_loop(0, c16, idx_body, 0)

        cps = [
            pltpu.async_copy(bg_hbm.at[bg_i], bg_v, sem),
            pltpu.async_copy(meta_hbm.at[me_i], me_v, sem),
            pltpu.async_copy(pers_hbm.at[pe_i], pe_v, sem),
        ]
        for cp in cps:
            cp.wait()

        def sum_body(c, carry):
            off = pl.multiple_of(c * 16, 16)
            bg_v[pl.ds(off, 16)] = (bg_v[pl.ds(off, 16)]
                                    + me_v[pl.ds(off, 16)]
                                    + pe_v[pl.ds(off, 16)])
            return carry
        lax.fori_loop(0, c16, sum_body, 0)

        pltpu.sync_copy(bg_v, out_hbm.at[pl.ds(en, n_per_w)])

    return k(node_flat, b_of_i, r32, m32, p32, bg_flat, meta_flat, pers_flat)


def _tc_logsigmoid_reduce(z, signs, paths, pad):
    B, L = z.shape
    blk = 1024

    def body(z_ref, s_ref, p_ref, o_ref):
        x = s_ref[...] * z_ref[...]
        lp = jnp.minimum(x, 0.0) - jnp.log1p(jnp.exp(-jnp.abs(x)))
        lp = jnp.where(p_ref[...] != pad, lp, 0.0)
        o_ref[...] = jnp.sum(lp, axis=1)

    return pl.pallas_call(
        body,
        grid=(B // blk,),
        in_specs=[
            pl.BlockSpec((blk, L), lambda i: (i, 0)),
            pl.BlockSpec((blk, L), lambda i: (i, 0)),
            pl.BlockSpec((blk, L), lambda i: (i, 0)),
        ],
        out_specs=pl.BlockSpec((blk,), lambda i: (i,)),
        out_shape=jax.ShapeDtypeStruct((B,), jnp.float32),
    )(z, signs, paths)


def kernel(m_idx, p_idx, r_idx, node_paths, node_signs, eta_bg, eta_meta, eta_pers):
    B, L = node_paths.shape
    R, V = eta_bg.shape
    paths32 = node_paths.astype(jnp.int32)
    n_per_w = (B // NW) * L
    b_of_i = (jnp.arange(n_per_w, dtype=jnp.int32) // L).astype(jnp.int32)
    z = _sc_gather_logits(
        paths32.reshape(-1),
        b_of_i,
        r_idx.astype(jnp.int32),
        m_idx.astype(jnp.int32),
        p_idx.astype(jnp.int32),
        eta_bg.reshape(-1),
        eta_meta.reshape(-1),
        eta_pers.reshape(-1),
        B, L, R, V)
    return _tc_logsigmoid_reduce(z.reshape(B, L), node_signs, paths32, V - 1)


# speedup vs baseline: 10.4602x; 10.4185x over previous
import functools

import jax
import jax.numpy as jnp
from jax import lax
from jax.experimental import pallas as pl
from jax.experimental.pallas import tpu as pltpu
from jax.experimental.pallas import tpu_sc as plsc

NC = 2
NS = 16
NW = NC * NS


def _sc_gather_logits(node_flat, b_of_i, r32, m32, p32, bg_flat, meta_flat,
                      pers_flat, B, L, R, TPB):
    b_per_w = B // NW
    n_per_w = b_per_w * L
    c16 = n_per_w // 16
    cdma = n_per_w // 128
    KW = 4

    mesh = plsc.VectorSubcoreMesh(core_axis_name="c", subcore_axis_name="s",
                                  num_cores=NC, num_subcores=NS)

    @functools.partial(
        pl.kernel,
        out_type=jax.ShapeDtypeStruct((B * L,), jnp.float32),
        mesh=mesh,
        compiler_params=pltpu.CompilerParams(needs_layout_passes=False),
        scratch_types=[
            pltpu.VMEM((n_per_w,), jnp.int32),
            pltpu.VMEM((n_per_w,), jnp.int32),
            pltpu.VMEM((b_per_w,), jnp.int32),
            pltpu.VMEM((b_per_w,), jnp.int32),
            pltpu.VMEM((b_per_w,), jnp.int32),
            pltpu.VMEM((b_per_w,), jnp.int32),
            pltpu.VMEM((b_per_w,), jnp.int32),
            pltpu.VMEM((b_per_w,), jnp.int32),
            pltpu.VMEM((n_per_w,), jnp.int32),
            pltpu.VMEM((n_per_w,), jnp.int32),
            pltpu.VMEM((n_per_w,), jnp.int32),
            pltpu.VMEM((n_per_w,), jnp.float32),
            pltpu.VMEM((n_per_w,), jnp.float32),
            pltpu.VMEM((n_per_w,), jnp.float32),
            pltpu.SemaphoreType.DMA,
        ],
    )
    def k(node_hbm, b_hbm, r_hbm, m_hbm, p_hbm, bg_hbm, meta_hbm, pers_hbm,
          out_hbm, b_v, node_v, r_v, m_v, p_v, bgb_v, mb_v, pb_v,
          bg_i, me_i, pe_i, bg_v, me_v, pe_v, sem):
        wid = lax.axis_index("s") * NC + lax.axis_index("c")
        eb = pl.multiple_of(wid * b_per_w, 8)
        en = pl.multiple_of(wid * n_per_w, 8)

        pltpu.sync_copy(b_hbm, b_v)
        pltpu.sync_copy(node_hbm.at[pl.ds(en, n_per_w)], node_v)
        pltpu.sync_copy(r_hbm.at[pl.ds(eb, b_per_w)], r_v)
        pltpu.sync_copy(m_hbm.at[pl.ds(eb, b_per_w)], m_v)
        pltpu.sync_copy(p_hbm.at[pl.ds(eb, b_per_w)], p_v)

        def base_body(j, carry):
            off = pl.multiple_of(j * 16, 16)
            r16 = r_v[pl.ds(off, 16)]
            m16 = m_v[pl.ds(off, 16)]
            p16 = p_v[pl.ds(off, 16)]
            gm = m16 * R + r16
            gp = p16 * R + r16
            bgb_v[pl.ds(off, 16)] = (r16 >> 3) * TPB + (r16 & 7) * 128
            mb_v[pl.ds(off, 16)] = (gm >> 3) * TPB + (gm & 7) * 128
            pb_v[pl.ds(off, 16)] = (gp >> 3) * TPB + (gp & 7) * 128
            return carry
        lax.fori_loop(0, b_per_w // 16, base_body, 0)

        def idx_body(c, carry):
            off = pl.multiple_of(c * 16, 16)
            b16 = b_v[pl.ds(off, 16)]
            node16 = node_v[pl.ds(off, 16)]
            vphys = node16 + (node16 >> 7) * 896
            bgb = plsc.load_gather(bgb_v, [b16])
            mb = plsc.load_gather(mb_v, [b16])
            pb = plsc.load_gather(pb_v, [b16])
            bg_i[pl.ds(off, 16)] = bgb + vphys
            me_i[pl.ds(off, 16)] = mb + vphys
            pe_i[pl.ds(off, 16)] = pb + vphys
            return carry
        lax.fori_loop(0, c16, idx_body, 0)

        cps = [
            pltpu.async_copy(bg_hbm.at[bg_i], bg_v, sem),
            pltpu.async_copy(meta_hbm.at[me_i], me_v, sem),
            pltpu.async_copy(pers_hbm.at[pe_i], pe_v, sem),
        ]
        for cp in cps:
            cp.wait()

        def sum_body(c, carry):
            off = pl.multiple_of(c * 16, 16)
            bg_v[pl.ds(off, 16)] = (bg_v[pl.ds(off, 16)]
                                    + me_v[pl.ds(off, 16)]
                                    + pe_v[pl.ds(off, 16)])
            return carry
        lax.fori_loop(0, c16, sum_body, 0)

        pltpu.sync_copy(bg_v, out_hbm.at[pl.ds(en, n_per_w)])

    return k(node_flat, b_of_i, r32, m32, p32, bg_flat, meta_flat, pers_flat)


def _tc_logsigmoid_reduce(z, signs, paths, pad):
    B, L = z.shape
    blk = 1024

    def body(z_ref, s_ref, p_ref, o_ref):
        x = s_ref[...] * z_ref[...]
        lp = jnp.minimum(x, 0.0) - jnp.log1p(jnp.exp(-jnp.abs(x)))
        lp = jnp.where(p_ref[...] != pad, lp, 0.0)
        o_ref[...] = jnp.sum(lp, axis=1)

    return pl.pallas_call(
        body,
        grid=(B // blk,),
        in_specs=[
            pl.BlockSpec((blk, L), lambda i: (i, 0)),
            pl.BlockSpec((blk, L), lambda i: (i, 0)),
            pl.BlockSpec((blk, L), lambda i: (i, 0)),
        ],
        out_specs=pl.BlockSpec((blk,), lambda i: (i,)),
        out_shape=jax.ShapeDtypeStruct((B,), jnp.float32),
    )(z, signs, paths)


def _tile_major_flat(table2d):
    rows, V = table2d.shape
    vt = (V + 127) // 128
    padded = jnp.pad(table2d, ((0, 0), (0, vt * 128 - V)))
    y = padded.reshape(rows // 8, 8, vt, 128).transpose(0, 2, 1, 3)
    return y.reshape(-1)


def kernel(m_idx, p_idx, r_idx, node_paths, node_signs, eta_bg, eta_meta, eta_pers):
    B, L = node_paths.shape
    R, V = eta_bg.shape
    M = eta_meta.shape[0]
    P = eta_pers.shape[0]
    TPB = 8 * ((V + 127) // 128) * 128
    paths32 = node_paths.astype(jnp.int32)
    n_per_w = (B // NW) * L
    b_of_i = (jnp.arange(n_per_w, dtype=jnp.int32) // L).astype(jnp.int32)
    z = _sc_gather_logits(
        paths32.reshape(-1),
        b_of_i,
        r_idx.astype(jnp.int32),
        m_idx.astype(jnp.int32),
        p_idx.astype(jnp.int32),
        _tile_major_flat(eta_bg),
        _tile_major_flat(eta_meta.reshape(M * R, V)),
        _tile_major_flat(eta_pers.reshape(P * R, V)),
        B, L, R, TPB)
    return _tc_logsigmoid_reduce(z.reshape(B, L), node_signs, paths32, V - 1)


# speedup vs baseline: 11.0851x; 1.0597x over previous
import functools

import jax
import jax.numpy as jnp
from jax import lax
from jax.experimental import pallas as pl
from jax.experimental.pallas import tpu as pltpu
from jax.experimental.pallas import tpu_sc as plsc

NC = 2
NS = 16
NW = NC * NS


def _sc_gather_bg_meta(node_flat, b_of_i, r32, m32, p32, bg_flat, meta_flat,
                       B, L, R, TPB):
    b_per_w = B // NW
    n_per_w = b_per_w * L
    c16 = n_per_w // 16

    mesh = plsc.VectorSubcoreMesh(core_axis_name="c", subcore_axis_name="s",
                                  num_cores=NC, num_subcores=NS)

    @functools.partial(
        pl.kernel,
        out_type=(jax.ShapeDtypeStruct((B * L,), jnp.float32),
                  jax.ShapeDtypeStruct((B * L,), jnp.int32)),
        mesh=mesh,
        compiler_params=pltpu.CompilerParams(needs_layout_passes=False),
        scratch_types=[
            pltpu.VMEM((n_per_w,), jnp.int32),
            pltpu.VMEM((n_per_w,), jnp.int32),
            pltpu.VMEM((b_per_w,), jnp.int32),
            pltpu.VMEM((b_per_w,), jnp.int32),
            pltpu.VMEM((b_per_w,), jnp.int32),
            pltpu.VMEM((b_per_w,), jnp.int32),
            pltpu.VMEM((b_per_w,), jnp.int32),
            pltpu.VMEM((b_per_w,), jnp.int32),
            pltpu.VMEM((n_per_w,), jnp.int32),
            pltpu.VMEM((n_per_w,), jnp.int32),
            pltpu.VMEM((n_per_w,), jnp.int32),
            pltpu.VMEM((n_per_w,), jnp.float32),
            pltpu.VMEM((n_per_w,), jnp.float32),
            pltpu.SemaphoreType.DMA,
        ],
    )
    def k(node_hbm, b_hbm, r_hbm, m_hbm, p_hbm, bg_hbm, meta_hbm,
          z1_hbm, pidx_hbm, b_v, node_v, r_v, m_v, p_v, bgb_v, mb_v, pb_v,
          bg_i, me_i, pe_i, bg_v, me_v, sem):
        wid = lax.axis_index("s") * NC + lax.axis_index("c")
        eb = pl.multiple_of(wid * b_per_w, 8)
        en = pl.multiple_of(wid * n_per_w, 8)

        pltpu.sync_copy(b_hbm, b_v)
        pltpu.sync_copy(node_hbm.at[pl.ds(en, n_per_w)], node_v)
        pltpu.sync_copy(r_hbm.at[pl.ds(eb, b_per_w)], r_v)
        pltpu.sync_copy(m_hbm.at[pl.ds(eb, b_per_w)], m_v)
        pltpu.sync_copy(p_hbm.at[pl.ds(eb, b_per_w)], p_v)

        def base_body(j, carry):
            off = pl.multiple_of(j * 16, 16)
            r16 = r_v[pl.ds(off, 16)]
            m16 = m_v[pl.ds(off, 16)]
            p16 = p_v[pl.ds(off, 16)]
            gm = m16 * R + r16
            gp = p16 * R + r16
            bgb_v[pl.ds(off, 16)] = (r16 >> 3) * TPB + (r16 & 7) * 128
            mb_v[pl.ds(off, 16)] = (gm >> 3) * TPB + (gm & 7) * 128
            pb_v[pl.ds(off, 16)] = (gp >> 3) * TPB + (gp & 7) * 128
            return carry
        lax.fori_loop(0, b_per_w // 16, base_body, 0)

        def idx_body(c, carry):
            off = pl.multiple_of(c * 16, 16)
            b16 = b_v[pl.ds(off, 16)]
            node16 = node_v[pl.ds(off, 16)]
            vphys = node16 + (node16 >> 7) * 896
            bgb = plsc.load_gather(bgb_v, [b16])
            mb = plsc.load_gather(mb_v, [b16])
            pb = plsc.load_gather(pb_v, [b16])
            bg_i[pl.ds(off, 16)] = bgb + vphys
            me_i[pl.ds(off, 16)] = mb + vphys
            pe_i[pl.ds(off, 16)] = pb + vphys
            return carry
        lax.fori_loop(0, c16, idx_body, 0)

        cps = [
            pltpu.async_copy(bg_hbm.at[bg_i], bg_v, sem),
            pltpu.async_copy(meta_hbm.at[me_i], me_v, sem),
            pltpu.async_copy(pe_i, pidx_hbm.at[pl.ds(en, n_per_w)], sem),
        ]
        for cp in cps:
            cp.wait()

        def sum_body(c, carry):
            off = pl.multiple_of(c * 16, 16)
            bg_v[pl.ds(off, 16)] = (bg_v[pl.ds(off, 16)]
                                    + me_v[pl.ds(off, 16)])
            return carry
        lax.fori_loop(0, c16, sum_body, 0)

        pltpu.sync_copy(bg_v, z1_hbm.at[pl.ds(en, n_per_w)])

    return k(node_flat, b_of_i, r32, m32, p32, bg_flat, meta_flat)


def _sc_gather_pers(z1, pidx, pers_flat, B, L):
    n_per_w = (B // NW) * L
    c16 = n_per_w // 16

    mesh = plsc.VectorSubcoreMesh(core_axis_name="c", subcore_axis_name="s",
                                  num_cores=NC, num_subcores=NS)

    @functools.partial(
        pl.kernel,
        out_type=jax.ShapeDtypeStruct((B * L,), jnp.float32),
        mesh=mesh,
        compiler_params=pltpu.CompilerParams(needs_layout_passes=False),
        scratch_types=[
            pltpu.VMEM((n_per_w,), jnp.int32),
            pltpu.VMEM((n_per_w,), jnp.float32),
            pltpu.VMEM((n_per_w,), jnp.float32),
            pltpu.SemaphoreType.DMA,
        ],
    )
    def k(z1_hbm, pidx_hbm, pers_hbm, out_hbm, pe_i, z1_v, pe_v, sem):
        wid = lax.axis_index("s") * NC + lax.axis_index("c")
        en = pl.multiple_of(wid * n_per_w, 8)

        pltpu.sync_copy(pidx_hbm.at[pl.ds(en, n_per_w)], pe_i)
        cps = [
            pltpu.async_copy(pers_hbm.at[pe_i], pe_v, sem),
            pltpu.async_copy(z1_hbm.at[pl.ds(en, n_per_w)], z1_v, sem),
        ]
        for cp in cps:
            cp.wait()

        def sum_body(c, carry):
            off = pl.multiple_of(c * 16, 16)
            z1_v[pl.ds(off, 16)] = (z1_v[pl.ds(off, 16)]
                                    + pe_v[pl.ds(off, 16)])
            return carry
        lax.fori_loop(0, c16, sum_body, 0)

        pltpu.sync_copy(z1_v, out_hbm.at[pl.ds(en, n_per_w)])

    return k(z1, pidx, pers_flat)


def _tc_logsigmoid_reduce(z, signs, paths, pad):
    B, L = z.shape
    blk = 1024

    def body(z_ref, s_ref, p_ref, o_ref):
        x = s_ref[...] * z_ref[...]
        lp = jnp.minimum(x, 0.0) - jnp.log1p(jnp.exp(-jnp.abs(x)))
        lp = jnp.where(p_ref[...] != pad, lp, 0.0)
        o_ref[...] = jnp.sum(lp, axis=1)

    return pl.pallas_call(
        body,
        grid=(B // blk,),
        in_specs=[
            pl.BlockSpec((blk, L), lambda i: (i, 0)),
            pl.BlockSpec((blk, L), lambda i: (i, 0)),
            pl.BlockSpec((blk, L), lambda i: (i, 0)),
        ],
        out_specs=pl.BlockSpec((blk,), lambda i: (i,)),
        out_shape=jax.ShapeDtypeStruct((B,), jnp.float32),
    )(z, signs, paths)


def _tile_major_flat(table2d):
    rows, V = table2d.shape
    vt = (V + 127) // 128
    padded = jnp.pad(table2d, ((0, 0), (0, vt * 128 - V)))
    y = padded.reshape(rows // 8, 8, vt, 128).transpose(0, 2, 1, 3)
    return y.reshape(-1)


def kernel(m_idx, p_idx, r_idx, node_paths, node_signs, eta_bg, eta_meta, eta_pers):
    B, L = node_paths.shape
    R, V = eta_bg.shape
    M = eta_meta.shape[0]
    P = eta_pers.shape[0]
    TPB = 8 * ((V + 127) // 128) * 128
    paths32 = node_paths.astype(jnp.int32)
    n_per_w = (B // NW) * L
    b_of_i = (jnp.arange(n_per_w, dtype=jnp.int32) // L).astype(jnp.int32)
    z1, pidx = _sc_gather_bg_meta(
        paths32.reshape(-1),
        b_of_i,
        r_idx.astype(jnp.int32),
        m_idx.astype(jnp.int32),
        p_idx.astype(jnp.int32),
        _tile_major_flat(eta_bg),
        _tile_major_flat(eta_meta.reshape(M * R, V)),
        B, L, R, TPB)
    z = _sc_gather_pers(z1, pidx, _tile_major_flat(eta_pers.reshape(P * R, V)),
                        B, L)
    return _tc_logsigmoid_reduce(z.reshape(B, L), node_signs, paths32, V - 1)
